# baseline (speedup 1.0000x reference)
import functools

import jax
import jax.numpy as jnp
from jax import lax
from jax.experimental import pallas as pl
from jax.experimental.pallas import tpu as pltpu
from jax.experimental.pallas import tpu_sc as plsc


S_BLK = 512


def _tc_body(x_ref, pe_ref, out_ref):
    out_ref[...] = x_ref[...] + pe_ref[...][None, :, :]


def _tc_kernel(x, pos_embed):
    B, S, D = x.shape
    grid = (S // S_BLK,)
    return pl.pallas_call(
        _tc_body,
        grid=grid,
        in_specs=[
            pl.BlockSpec((B, S_BLK, D), lambda i: (0, i, 0)),
            pl.BlockSpec((S_BLK, D), lambda i: (i, 0)),
        ],
        out_specs=pl.BlockSpec((B, S_BLK, D), lambda i: (0, i, 0)),
        out_shape=jax.ShapeDtypeStruct((B, S, D), x.dtype),
    )(x, pos_embed[:S])


_NC, _NS, _L = 2, 16, 16
_NW = _NC * _NS

_B, _S, _D = 4, 8192, 1024
_ROWS_PER_W = _S // _NW
_CH = 16
_CHE = _CH * _D
_NCHUNK = _ROWS_PER_W // _CH


def _sc_body(x_hbm, pe_hbm, out_hbm, pe_v, x_v):
    wid = lax.axis_index("s") * _NC + lax.axis_index("c")
    s_base = wid * _ROWS_PER_W

    @pl.loop(0, _NCHUNK)
    def _chunk(c):
        pe_off = (s_base + c * _CH) * _D
        pltpu.sync_copy(pe_hbm.at[pl.ds(pe_off, _CHE)], pe_v)

        @pl.loop(0, _B)
        def _batch(b):
            x_off = b * (_S * _D) + pe_off
            pltpu.sync_copy(x_hbm.at[pl.ds(x_off, _CHE)], x_v)

            @plsc.parallel_loop(0, _CHE // _L, unroll=8)
            def _add(i):
                sl = pl.ds(i * _L, _L)
                x_v[sl] = x_v[sl] + pe_v[sl]

            pltpu.sync_copy(x_v, out_hbm.at[pl.ds(x_off, _CHE)])


def _sc_kernel(x, pos_embed):
    B, S, D = x.shape
    mesh = plsc.VectorSubcoreMesh(
        core_axis_name="c", subcore_axis_name="s",
        num_cores=_NC, num_subcores=_NS,
    )
    run = pl.kernel(
        _sc_body,
        out_type=jax.ShapeDtypeStruct((B * S * D,), x.dtype),
        mesh=mesh,
        scratch_types=[
            pltpu.VMEM((_CHE,), jnp.float32),
            pltpu.VMEM((_CHE,), jnp.float32),
        ],
    )
    out = run(x.reshape(-1), pos_embed[:S].reshape(-1))
    return out.reshape(B, S, D)


def kernel(x, pos_embed):
    return _sc_kernel(x, pos_embed)


# speedup vs baseline: 1.2903x; 1.2903x over previous
import functools

import jax
import jax.numpy as jnp
from jax import lax
from jax.experimental import pallas as pl
from jax.experimental.pallas import tpu as pltpu
from jax.experimental.pallas import tpu_sc as plsc


S_BLK = 512


def _tc_body(x_ref, pe_ref, out_ref):
    out_ref[...] = x_ref[...] + pe_ref[...][None, :, :]


def _tc_kernel(x, pos_embed):
    B, S, D = x.shape
    grid = (S // S_BLK,)
    return pl.pallas_call(
        _tc_body,
        grid=grid,
        in_specs=[
            pl.BlockSpec((B, S_BLK, D), lambda i: (0, i, 0)),
            pl.BlockSpec((S_BLK, D), lambda i: (i, 0)),
        ],
        out_specs=pl.BlockSpec((B, S_BLK, D), lambda i: (0, i, 0)),
        out_shape=jax.ShapeDtypeStruct((B, S, D), x.dtype),
    )(x, pos_embed[:S])


_NC, _NS, _L = 2, 16, 16
_NW = _NC * _NS

_B, _S, _D = 4, 8192, 1024
_ROWS_PER_W = _S // _NW
_CH = 16
_CHE = _CH * _D
_NCHUNK = _ROWS_PER_W // _CH


_NSTEP = _NCHUNK * _B


def _sc_body(x_hbm, pe_hbm, out_hbm,
             xi0, xi1, xo0, xo1, pe0, pe1,
             in_s0, in_s1, out_s0, out_s1, pe_s0, pe_s1):
    wid = lax.axis_index("s") * _NC + lax.axis_index("c")
    s_base = wid * _ROWS_PER_W
    xi = (xi0, xi1)
    xo = (xo0, xo1)
    pe = (pe0, pe1)
    in_s = (in_s0, in_s1)
    out_s = (out_s0, out_s1)
    pe_s = (pe_s0, pe_s1)

    def x_off(k):
        return (k & 3) * (_S * _D) + (s_base + (k >> 2) * _CH) * _D

    def pe_off(c):
        return (s_base + c * _CH) * _D

    pltpu.async_copy(x_hbm.at[pl.ds(x_off(0), _CHE)], xi[0], in_s[0])
    pltpu.async_copy(pe_hbm.at[pl.ds(pe_off(0), _CHE)], pe[0], pe_s[0])

    @pl.loop(0, _NCHUNK // 2)
    def _cc(cc):
        for q in (0, 1):
            c = cc * 2 + q
            for b in range(_B):
                k = c * _B + b
                p = (4 * q + b) & 1

                if b == 0:
                    pltpu.make_async_copy(
                        pe_hbm.at[pl.ds(0, _CHE)], pe[q], pe_s[q]).wait()

                    @pl.when(c < _NCHUNK - 1)
                    def _():
                        pltpu.async_copy(
                            pe_hbm.at[pl.ds(pe_off(c + 1), _CHE)],
                            pe[1 - q], pe_s[1 - q])

                @pl.when(k >= 2)
                def _():
                    pltpu.make_async_copy(
                        x_hbm.at[pl.ds(0, _CHE)], xo[p], out_s[p]).wait()

                pltpu.make_async_copy(
                    x_hbm.at[pl.ds(0, _CHE)], xi[p], in_s[p]).wait()

                @pl.when(k < _NSTEP - 1)
                def _():
                    pltpu.async_copy(
                        x_hbm.at[pl.ds(x_off(k + 1), _CHE)],
                        xi[1 - p], in_s[1 - p])

                xi_p, xo_p, pe_q = xi[p], xo[p], pe[q]

                @plsc.parallel_loop(0, _CHE // _L, unroll=8)
                def _add(i):
                    sl = pl.ds(i * _L, _L)
                    xo_p[sl] = xi_p[sl] + pe_q[sl]

                pltpu.async_copy(xo[p], out_hbm.at[pl.ds(x_off(k), _CHE)],
                                 out_s[p])

    pltpu.make_async_copy(x_hbm.at[pl.ds(0, _CHE)], xo[0], out_s[0]).wait()
    pltpu.make_async_copy(x_hbm.at[pl.ds(0, _CHE)], xo[1], out_s[1]).wait()


def _sc_kernel(x, pos_embed):
    B, S, D = x.shape
    mesh = plsc.VectorSubcoreMesh(
        core_axis_name="c", subcore_axis_name="s",
        num_cores=_NC, num_subcores=_NS,
    )
    run = pl.kernel(
        _sc_body,
        out_type=jax.ShapeDtypeStruct((B * S * D,), x.dtype),
        mesh=mesh,
        scratch_types=(
            [pltpu.VMEM((_CHE,), jnp.float32) for _ in range(6)]
            + [pltpu.SemaphoreType.DMA for _ in range(6)]
        ),
    )
    out = run(x.reshape(-1), pos_embed[:S].reshape(-1))
    return out.reshape(B, S, D)


def kernel(x, pos_embed):
    return _sc_kernel(x, pos_embed)
